# baseline (speedup 1.0000x reference)
import functools

import jax
import jax.numpy as jnp
from jax.experimental import pallas as pl
from jax.experimental.pallas import tpu as pltpu

D = 256
DE = 16


def _msg_kernel(x_gath_ref, ea_ref, we_ref, be_ref, out_ref):
    emb = jnp.dot(ea_ref[...], we_ref[...], preferred_element_type=jnp.float32)
    out_ref[...] = jnp.maximum(x_gath_ref[...] + emb + be_ref[...], 0.0)


def _mlp_kernel(x_ref, agg_ref, w1_ref, b1_ref, w2_ref, b2_ref, eps_ref, out_ref):
    h = (1.0 + eps_ref[0]) * x_ref[...] + agg_ref[...]
    h1 = jnp.maximum(
        jnp.dot(h, w1_ref[...], preferred_element_type=jnp.float32) + b1_ref[...], 0.0
    )
    out_ref[...] = (
        jnp.dot(h1, w2_ref[...], preferred_element_type=jnp.float32) + b2_ref[...]
    )


def _gine_one(x, edge_index, edge_attr, We, be, W1, b1, W2, b2, eps):
    E = edge_attr.shape[0]
    N = x.shape[0]

    x_gath = jnp.take(x, edge_index[0], axis=0)

    EBLK = 3200
    msg = pl.pallas_call(
        _msg_kernel,
        grid=(E // EBLK,),
        in_specs=[
            pl.BlockSpec((EBLK, D), lambda i: (i, 0)),
            pl.BlockSpec((EBLK, DE), lambda i: (i, 0)),
            pl.BlockSpec((DE, D), lambda i: (0, 0)),
            pl.BlockSpec((1, D), lambda i: (0, 0)),
        ],
        out_specs=pl.BlockSpec((EBLK, D), lambda i: (i, 0)),
        out_shape=jax.ShapeDtypeStruct((E, D), jnp.float32),
    )(x_gath, edge_attr, We, be.reshape(1, D))

    agg = jax.ops.segment_sum(msg, edge_index[1], num_segments=N)

    NBLK = 1000
    out = pl.pallas_call(
        _mlp_kernel,
        grid=(N // NBLK,),
        in_specs=[
            pl.BlockSpec((NBLK, D), lambda i: (i, 0)),
            pl.BlockSpec((NBLK, D), lambda i: (i, 0)),
            pl.BlockSpec((D, D), lambda i: (0, 0)),
            pl.BlockSpec((1, D), lambda i: (0, 0)),
            pl.BlockSpec((D, D), lambda i: (0, 0)),
            pl.BlockSpec((1, D), lambda i: (0, 0)),
            pl.BlockSpec(memory_space=pltpu.SMEM),
        ],
        out_specs=pl.BlockSpec((NBLK, D), lambda i: (i, 0)),
        out_shape=jax.ShapeDtypeStruct((N, D), jnp.float32),
    )(x, agg, W1, b1.reshape(1, D), W2, b2.reshape(1, D), eps.reshape(1))

    return out


@jax.jit
def kernel(x_user, x_item, edge_index_to, edge_attr_to, edge_index_rev,
           edge_attr_rev, We_to, be_to, W1_to, b1_to, W2_to, b2_to, eps_to,
           We_rev, be_rev, W1_rev, b1_rev, W2_rev, b2_rev, eps_rev):
    out_item = _gine_one(x_user, edge_index_to, edge_attr_to,
                         We_to, be_to, W1_to, b1_to, W2_to, b2_to, eps_to)
    out_user = _gine_one(x_item, edge_index_rev, edge_attr_rev,
                         We_rev, be_rev, W1_rev, b1_rev, W2_rev, b2_rev, eps_rev)
    return (out_user, out_item)


# speedup vs baseline: 1.8446x; 1.8446x over previous
import functools

import jax
import jax.numpy as jnp
from jax import lax
from jax.experimental import pallas as pl
from jax.experimental.pallas import tpu as pltpu
from jax.experimental.pallas import tpu_sc as plsc

D = 256
DE = 16
DH = 128
NC = 2
NS = 16
K = 80


def _emb_kernel(ea_ref, we_ref, be_ref, out_ref):
    emb = (
        jnp.dot(ea_ref[...], we_ref[...], preferred_element_type=jnp.float32)
        + be_ref[...]
    )
    out_ref[0] = emb[:, :DH]
    out_ref[1] = emb[:, DH:]


def _mlp_kernel(x_ref, agg_ref, w1_ref, b1_ref, w2_ref, b2_ref, eps_ref, out_ref):
    agg = jnp.concatenate([agg_ref[0], agg_ref[1]], axis=-1)
    h = (1.0 + eps_ref[0]) * x_ref[...] + agg
    h1 = jnp.maximum(
        jnp.dot(h, w1_ref[...], preferred_element_type=jnp.float32) + b1_ref[...], 0.0
    )
    out_ref[...] = (
        jnp.dot(h1, w2_ref[...], preferred_element_type=jnp.float32) + b2_ref[...]
    )


def _sc_body(x2, srcs, dsts, emb2, out, src_v, dst_v, xrows_v, emb_v, agg_sh,
             sem):
    E = srcs.shape[0]
    N = x2.shape[1]
    c = lax.axis_index("c")
    s = lax.axis_index("s")
    chunks = E // NS // K
    nblocks = N // K
    bper = (nblocks + NS - 1) // NS

    zero16 = jnp.zeros((16,), jnp.float32)

    def zrow(r, _):
        for j in range(DH // 16):
            emb_v[r, pl.ds(j * 16, 16)] = zero16
        return 0

    lax.fori_loop(0, K, zrow, 0)

    for k in range(bper):
        b = s + k * NS

        @pl.when(b < nblocks)
        def _():
            pltpu.sync_copy(emb_v, agg_sh.at[pl.ds(b * K, K)])

    plsc.subcore_barrier()

    def chunk(i, _):
        base = s * (E // NS) + i * K
        pltpu.sync_copy(srcs.at[pl.ds(base, K)], src_v)
        pltpu.sync_copy(dsts.at[pl.ds(base, K)], dst_v)
        pltpu.async_copy(x2.at[c].at[src_v], xrows_v, sem).wait()
        pltpu.sync_copy(emb2.at[c].at[pl.ds(base, K)], emb_v)

        def row(r, _):
            for j in range(DH // 16):
                sl = pl.ds(j * 16, 16)
                emb_v[r, sl] = jnp.maximum(emb_v[r, sl] + xrows_v[r, sl], 0.0)
            return 0

        lax.fori_loop(0, K, row, 0)
        pltpu.sync_copy(emb_v, agg_sh.at[dst_v], add=True)
        return 0

    lax.fori_loop(0, chunks, chunk, 0)
    plsc.subcore_barrier()

    for k in range(bper):
        b = s + k * NS

        @pl.when(b < nblocks)
        def _():
            pltpu.sync_copy(agg_sh.at[pl.ds(b * K, K)], emb_v)
            pltpu.sync_copy(emb_v, out.at[c].at[pl.ds(b * K, K)])


def _sc_gather_scatter(x2, srcs, dsts, emb2):
    N = x2.shape[1]
    mesh = plsc.VectorSubcoreMesh(
        core_axis_name="c", subcore_axis_name="s", num_cores=NC, num_subcores=NS
    )
    return pl.kernel(
        _sc_body,
        out_type=jax.ShapeDtypeStruct((NC, N, DH), jnp.float32),
        mesh=mesh,
        scratch_types=[
            pltpu.VMEM((K,), jnp.int32),
            pltpu.VMEM((K,), jnp.int32),
            pltpu.VMEM((K, DH), jnp.float32),
            pltpu.VMEM((K, DH), jnp.float32),
            pltpu.VMEM_SHARED((N, DH), jnp.float32),
            pltpu.SemaphoreType.DMA,
        ],
    )(x2, srcs, dsts, emb2)


def _gine_one(x, x2, edge_index, edge_attr, We, be, W1, b1, W2, b2, eps):
    E = edge_attr.shape[0]
    N = x.shape[0]

    EBLK = 3200
    emb2 = pl.pallas_call(
        _emb_kernel,
        grid=(E // EBLK,),
        in_specs=[
            pl.BlockSpec((EBLK, DE), lambda i: (i, 0)),
            pl.BlockSpec((DE, D), lambda i: (0, 0)),
            pl.BlockSpec((1, D), lambda i: (0, 0)),
        ],
        out_specs=pl.BlockSpec((NC, EBLK, DH), lambda i: (0, i, 0)),
        out_shape=jax.ShapeDtypeStruct((NC, E, DH), jnp.float32),
    )(edge_attr, We, be.reshape(1, D))

    agg2 = _sc_gather_scatter(x2, edge_index[0], edge_index[1], emb2)

    NBLK = 1000
    out = pl.pallas_call(
        _mlp_kernel,
        grid=(N // NBLK,),
        in_specs=[
            pl.BlockSpec((NBLK, D), lambda i: (i, 0)),
            pl.BlockSpec((NC, NBLK, DH), lambda i: (0, i, 0)),
            pl.BlockSpec((D, D), lambda i: (0, 0)),
            pl.BlockSpec((1, D), lambda i: (0, 0)),
            pl.BlockSpec((D, D), lambda i: (0, 0)),
            pl.BlockSpec((1, D), lambda i: (0, 0)),
            pl.BlockSpec(memory_space=pltpu.SMEM),
        ],
        out_specs=pl.BlockSpec((NBLK, D), lambda i: (i, 0)),
        out_shape=jax.ShapeDtypeStruct((N, D), jnp.float32),
    )(x, agg2, W1, b1.reshape(1, D), W2, b2.reshape(1, D), eps.reshape(1))

    return out


@jax.jit
def kernel(x_user, x_item, edge_index_to, edge_attr_to, edge_index_rev,
           edge_attr_rev, We_to, be_to, W1_to, b1_to, W2_to, b2_to, eps_to,
           We_rev, be_rev, W1_rev, b1_rev, W2_rev, b2_rev, eps_rev):
    x_user2 = jnp.stack([x_user[:, :DH], x_user[:, DH:]])
    x_item2 = jnp.stack([x_item[:, :DH], x_item[:, DH:]])
    ei_to = edge_index_to.astype(jnp.int32)
    ei_rev = edge_index_rev.astype(jnp.int32)
    out_item = _gine_one(x_user, x_user2, ei_to, edge_attr_to,
                         We_to, be_to, W1_to, b1_to, W2_to, b2_to, eps_to)
    out_user = _gine_one(x_item, x_item2, ei_rev, edge_attr_rev,
                         We_rev, be_rev, W1_rev, b1_rev, W2_rev, b2_rev, eps_rev)
    return (out_user, out_item)


# speedup vs baseline: 2.8354x; 1.5372x over previous
import functools

import jax
import jax.numpy as jnp
from jax import lax
from jax.experimental import pallas as pl
from jax.experimental.pallas import tpu as pltpu
from jax.experimental.pallas import tpu_sc as plsc

D = 256
DE = 16
DH = 128
NC = 2
NS = 16
K = 80


def _emb_kernel(ea_ref, we_ref, be_ref, out_ref):
    emb = (
        jnp.dot(ea_ref[...], we_ref[...], preferred_element_type=jnp.float32)
        + be_ref[...]
    )
    out_ref[0] = emb[:, :DH]
    out_ref[1] = emb[:, DH:]


def _mlp_kernel(x_ref, agg_ref, w1_ref, b1_ref, w2_ref, b2_ref, eps_ref, out_ref):
    agg = jnp.concatenate([agg_ref[0], agg_ref[1]], axis=-1)
    h = (1.0 + eps_ref[0]) * x_ref[...] + agg
    h1 = jnp.maximum(
        jnp.dot(h, w1_ref[...], preferred_element_type=jnp.float32) + b1_ref[...], 0.0
    )
    out_ref[...] = (
        jnp.dot(h1, w2_ref[...], preferred_element_type=jnp.float32) + b2_ref[...]
    )


def _sc_body(x2, srcs, dsts, emb2, out, src_v0, src_v1, dst_v0, dst_v1,
             xr0, xr1, em0, em1, agg_sh, sem):
    E = srcs.shape[0]
    N = x2.shape[1]
    c = lax.axis_index("c")
    s = lax.axis_index("s")
    ept = E // NS
    chunks = ept // K
    nblocks = N // K
    bper = (nblocks + NS - 1) // NS

    src_v = (src_v0, src_v1)
    dst_v = (dst_v0, dst_v1)
    xr = (xr0, xr1)
    em = (em0, em1)

    zero16 = jnp.zeros((16,), jnp.float32)

    def zrow(r, _):
        for j in range(DH // 16):
            em0[r, pl.ds(j * 16, 16)] = zero16
        return 0

    lax.fori_loop(0, K, zrow, 0)

    for k in range(bper):
        b = s + k * NS

        @pl.when(b < nblocks)
        def _():
            pltpu.sync_copy(em0, agg_sh.at[pl.ds(b * K, K)])

    plsc.subcore_barrier()

    def issue_in(i, b):
        base = s * ept + i * K
        pltpu.async_copy(srcs.at[pl.ds(base, K)], src_v[b], sem)
        pltpu.async_copy(dsts.at[pl.ds(base, K)], dst_v[b], sem)
        pltpu.async_copy(emb2.at[c].at[pl.ds(base, K)], em[b], sem)

    def wait_in(b):
        pltpu.make_async_copy(srcs.at[pl.ds(0, K)], src_v[b], sem).wait()
        pltpu.make_async_copy(dsts.at[pl.ds(0, K)], dst_v[b], sem).wait()
        pltpu.make_async_copy(emb2.at[c].at[pl.ds(0, K)], em[b], sem).wait()

    def issue_sc(b):
        pltpu.sync_copy(em[b], agg_sh.at[dst_v[b]], add=True)

    def wait_sc(b):
        pass

    def compute(b):
        emb_v = em[b]
        xr_v = xr[b]

        def row(r, _):
            for j in range(DH // 16):
                sl = pl.ds(j * 16, 16)
                emb_v[r, sl] = jnp.maximum(emb_v[r, sl] + xr_v[r, sl], 0.0)
            return 0

        lax.fori_loop(0, K, row, 0)

    issue_in(0, 0)

    def gather(b):
        pltpu.async_copy(x2.at[c].at[src_v[b]], xr[b], sem).wait()

    def outer(g, _):
        for b in range(2):
            i = 2 * g + b
            if b == 0:
                @pl.when(g >= 1)
                def _():
                    wait_sc(1)
            else:
                wait_sc(0)
            wait_in(b)
            gather(b)
            issue_in(i + 1, 1 - b)
            compute(b)
            issue_sc(b)
        return 0

    lax.fori_loop(0, (chunks - 1) // 2, outer, 0)

    wait_sc(1)
    wait_in(0)
    gather(0)
    compute(0)
    issue_sc(0)
    wait_sc(0)
    plsc.subcore_barrier()

    for k in range(bper):
        b = s + k * NS

        @pl.when(b < nblocks)
        def _():
            pltpu.sync_copy(agg_sh.at[pl.ds(b * K, K)], em0)
            pltpu.sync_copy(em0, out.at[c].at[pl.ds(b * K, K)])


def _sc_gather_scatter(x2, srcs, dsts, emb2):
    N = x2.shape[1]
    mesh = plsc.VectorSubcoreMesh(
        core_axis_name="c", subcore_axis_name="s", num_cores=NC, num_subcores=NS
    )
    return pl.kernel(
        _sc_body,
        out_type=jax.ShapeDtypeStruct((NC, N, DH), jnp.float32),
        mesh=mesh,
        scratch_types=[
            pltpu.VMEM((K,), jnp.int32),
            pltpu.VMEM((K,), jnp.int32),
            pltpu.VMEM((K,), jnp.int32),
            pltpu.VMEM((K,), jnp.int32),
            pltpu.VMEM((K, DH), jnp.float32),
            pltpu.VMEM((K, DH), jnp.float32),
            pltpu.VMEM((K, DH), jnp.float32),
            pltpu.VMEM((K, DH), jnp.float32),
            pltpu.VMEM_SHARED((N, DH), jnp.float32),
            pltpu.SemaphoreType.DMA,
        ],
    )(x2, srcs, dsts, emb2)


def _gine_one(x, x2, edge_index, edge_attr, We, be, W1, b1, W2, b2, eps):
    E = edge_attr.shape[0]
    N = x.shape[0]

    EBLK = 3200
    emb2 = pl.pallas_call(
        _emb_kernel,
        grid=(E // EBLK,),
        in_specs=[
            pl.BlockSpec((EBLK, DE), lambda i: (i, 0)),
            pl.BlockSpec((DE, D), lambda i: (0, 0)),
            pl.BlockSpec((1, D), lambda i: (0, 0)),
        ],
        out_specs=pl.BlockSpec((NC, EBLK, DH), lambda i: (0, i, 0)),
        out_shape=jax.ShapeDtypeStruct((NC, E, DH), jnp.float32),
    )(edge_attr, We, be.reshape(1, D))

    agg2 = _sc_gather_scatter(x2, edge_index[0], edge_index[1], emb2)

    NBLK = 1000
    out = pl.pallas_call(
        _mlp_kernel,
        grid=(N // NBLK,),
        in_specs=[
            pl.BlockSpec((NBLK, D), lambda i: (i, 0)),
            pl.BlockSpec((NC, NBLK, DH), lambda i: (0, i, 0)),
            pl.BlockSpec((D, D), lambda i: (0, 0)),
            pl.BlockSpec((1, D), lambda i: (0, 0)),
            pl.BlockSpec((D, D), lambda i: (0, 0)),
            pl.BlockSpec((1, D), lambda i: (0, 0)),
            pl.BlockSpec(memory_space=pltpu.SMEM),
        ],
        out_specs=pl.BlockSpec((NBLK, D), lambda i: (i, 0)),
        out_shape=jax.ShapeDtypeStruct((N, D), jnp.float32),
    )(x, agg2, W1, b1.reshape(1, D), W2, b2.reshape(1, D), eps.reshape(1))

    return out


@jax.jit
def kernel(x_user, x_item, edge_index_to, edge_attr_to, edge_index_rev,
           edge_attr_rev, We_to, be_to, W1_to, b1_to, W2_to, b2_to, eps_to,
           We_rev, be_rev, W1_rev, b1_rev, W2_rev, b2_rev, eps_rev):
    x_user2 = jnp.stack([x_user[:, :DH], x_user[:, DH:]])
    x_item2 = jnp.stack([x_item[:, :DH], x_item[:, DH:]])
    ei_to = edge_index_to.astype(jnp.int32)
    ei_rev = edge_index_rev.astype(jnp.int32)
    out_item = _gine_one(x_user, x_user2, ei_to, edge_attr_to,
                         We_to, be_to, W1_to, b1_to, W2_to, b2_to, eps_to)
    out_user = _gine_one(x_item, x_item2, ei_rev, edge_attr_rev,
                         We_rev, be_rev, W1_rev, b1_rev, W2_rev, b2_rev, eps_rev)
    return (out_user, out_item)


# speedup vs baseline: 3.2244x; 1.1372x over previous
import functools

import jax
import jax.numpy as jnp
from jax import lax
from jax.experimental import pallas as pl
from jax.experimental.pallas import tpu as pltpu
from jax.experimental.pallas import tpu_sc as plsc

D = 256
DE = 16
DH = 128
NC = 2
NS = 16
K = 80


def _emb_kernel(ea_ref, we_ref, be_ref, out_ref):
    emb = (
        jnp.dot(ea_ref[...], we_ref[...], preferred_element_type=jnp.float32)
        + be_ref[...]
    )
    out_ref[0] = emb[:, :DH]
    out_ref[1] = emb[:, DH:]


def _mlp_kernel(x_ref, agg_ref, w1_ref, b1_ref, w2_ref, b2_ref, eps_ref, out_ref):
    agg = jnp.concatenate([agg_ref[0], agg_ref[1]], axis=-1)
    h = (1.0 + eps_ref[0]) * x_ref[...] + agg
    h1 = jnp.maximum(
        jnp.dot(h, w1_ref[...], preferred_element_type=jnp.float32) + b1_ref[...], 0.0
    )
    out_ref[...] = (
        jnp.dot(h1, w2_ref[...], preferred_element_type=jnp.float32) + b2_ref[...]
    )


def _sc_body(x2, srcs, dsts, emb2, out, src_v0, src_v1, dst_v0, dst_v1,
             xr0, xr1, em0, em1, agg_sh, sin0, sin1, sgat0, sgat1):
    E = srcs.shape[0]
    N = x2.shape[1]
    c = lax.axis_index("c")
    s = lax.axis_index("s")
    ept = E // NS
    chunks = ept // K
    nblocks = N // K
    bper = (nblocks + NS - 1) // NS

    src_v = (src_v0, src_v1)
    dst_v = (dst_v0, dst_v1)
    xr = (xr0, xr1)
    em = (em0, em1)
    sin = (sin0, sin1)
    sgat = (sgat0, sgat1)

    zero16 = jnp.zeros((16,), jnp.float32)

    def zrow(r, _):
        for j in range(DH // 16):
            em0[r, pl.ds(j * 16, 16)] = zero16
        return 0

    lax.fori_loop(0, K, zrow, 0)

    for k in range(bper):
        b = s + k * NS

        @pl.when(b < nblocks)
        def _():
            pltpu.sync_copy(em0, agg_sh.at[pl.ds(b * K, K)])

    plsc.subcore_barrier()

    def issue_in(i, b):
        base = s * ept + i * K
        pltpu.async_copy(srcs.at[pl.ds(base, K)], src_v[b], sin[b])
        pltpu.async_copy(dsts.at[pl.ds(base, K)], dst_v[b], sin[b])
        pltpu.async_copy(emb2.at[c].at[pl.ds(base, K)], em[b], sin[b])

    def wait_in(b):
        pltpu.make_async_copy(srcs.at[pl.ds(0, K)], src_v[b], sin[b]).wait()
        pltpu.make_async_copy(dsts.at[pl.ds(0, K)], dst_v[b], sin[b]).wait()
        pltpu.make_async_copy(emb2.at[c].at[pl.ds(0, K)], em[b], sin[b]).wait()

    def issue_gat(b):
        pltpu.async_copy(x2.at[c].at[src_v[b]], xr[b], sgat[b])

    def wait_gat(b):
        pltpu.make_async_copy(x2.at[c].at[pl.ds(0, K)], xr[b], sgat[b]).wait()

    def scatter(b):
        pltpu.sync_copy(em[b], agg_sh.at[dst_v[b]], add=True)

    def compute(b, r0, r1):
        emb_v = em[b]
        xr_v = xr[b]

        def row(r, _):
            for j in range(DH // 16):
                sl = pl.ds(j * 16, 16)
                emb_v[r, sl] = jnp.maximum(emb_v[r, sl] + xr_v[r, sl], 0.0)
            return 0

        lax.fori_loop(r0, r1, row, 0)

    issue_in(0, 0)
    wait_in(0)
    issue_gat(0)

    def outer(g, _):
        for b in range(2):
            i = 2 * g + b
            issue_in(i + 1, 1 - b)
            wait_gat(b)
            compute(b, 0, K // 2)
            wait_in(1 - b)
            issue_gat(1 - b)
            compute(b, K // 2, K)
            scatter(b)
        return 0

    lax.fori_loop(0, (chunks - 1) // 2, outer, 0)

    wait_gat(0)
    compute(0, 0, K)
    scatter(0)
    plsc.subcore_barrier()

    for k in range(bper):
        b = s + k * NS

        @pl.when(b < nblocks)
        def _():
            pltpu.sync_copy(agg_sh.at[pl.ds(b * K, K)], em0)
            pltpu.sync_copy(em0, out.at[c].at[pl.ds(b * K, K)])


def _sc_gather_scatter(x2, srcs, dsts, emb2):
    N = x2.shape[1]
    mesh = plsc.VectorSubcoreMesh(
        core_axis_name="c", subcore_axis_name="s", num_cores=NC, num_subcores=NS
    )
    return pl.kernel(
        _sc_body,
        out_type=jax.ShapeDtypeStruct((NC, N, DH), jnp.float32),
        mesh=mesh,
        scratch_types=[
            pltpu.VMEM((K,), jnp.int32),
            pltpu.VMEM((K,), jnp.int32),
            pltpu.VMEM((K,), jnp.int32),
            pltpu.VMEM((K,), jnp.int32),
            pltpu.VMEM((K, DH), jnp.float32),
            pltpu.VMEM((K, DH), jnp.float32),
            pltpu.VMEM((K, DH), jnp.float32),
            pltpu.VMEM((K, DH), jnp.float32),
            pltpu.VMEM_SHARED((N, DH), jnp.float32),
            pltpu.SemaphoreType.DMA,
            pltpu.SemaphoreType.DMA,
            pltpu.SemaphoreType.DMA,
            pltpu.SemaphoreType.DMA,
        ],
    )(x2, srcs, dsts, emb2)


def _gine_one(x, x2, edge_index, edge_attr, We, be, W1, b1, W2, b2, eps):
    E = edge_attr.shape[0]
    N = x.shape[0]

    EBLK = 3200
    emb2 = pl.pallas_call(
        _emb_kernel,
        grid=(E // EBLK,),
        in_specs=[
            pl.BlockSpec((EBLK, DE), lambda i: (i, 0)),
            pl.BlockSpec((DE, D), lambda i: (0, 0)),
            pl.BlockSpec((1, D), lambda i: (0, 0)),
        ],
        out_specs=pl.BlockSpec((NC, EBLK, DH), lambda i: (0, i, 0)),
        out_shape=jax.ShapeDtypeStruct((NC, E, DH), jnp.float32),
    )(edge_attr, We, be.reshape(1, D))

    agg2 = _sc_gather_scatter(x2, edge_index[0], edge_index[1], emb2)

    NBLK = 1000
    out = pl.pallas_call(
        _mlp_kernel,
        grid=(N // NBLK,),
        in_specs=[
            pl.BlockSpec((NBLK, D), lambda i: (i, 0)),
            pl.BlockSpec((NC, NBLK, DH), lambda i: (0, i, 0)),
            pl.BlockSpec((D, D), lambda i: (0, 0)),
            pl.BlockSpec((1, D), lambda i: (0, 0)),
            pl.BlockSpec((D, D), lambda i: (0, 0)),
            pl.BlockSpec((1, D), lambda i: (0, 0)),
            pl.BlockSpec(memory_space=pltpu.SMEM),
        ],
        out_specs=pl.BlockSpec((NBLK, D), lambda i: (i, 0)),
        out_shape=jax.ShapeDtypeStruct((N, D), jnp.float32),
    )(x, agg2, W1, b1.reshape(1, D), W2, b2.reshape(1, D), eps.reshape(1))

    return out


@jax.jit
def kernel(x_user, x_item, edge_index_to, edge_attr_to, edge_index_rev,
           edge_attr_rev, We_to, be_to, W1_to, b1_to, W2_to, b2_to, eps_to,
           We_rev, be_rev, W1_rev, b1_rev, W2_rev, b2_rev, eps_rev):
    x_user2 = jnp.stack([x_user[:, :DH], x_user[:, DH:]])
    x_item2 = jnp.stack([x_item[:, :DH], x_item[:, DH:]])
    ei_to = edge_index_to.astype(jnp.int32)
    ei_rev = edge_index_rev.astype(jnp.int32)
    out_item = _gine_one(x_user, x_user2, ei_to, edge_attr_to,
                         We_to, be_to, W1_to, b1_to, W2_to, b2_to, eps_to)
    out_user = _gine_one(x_item, x_item2, ei_rev, edge_attr_rev,
                         We_rev, be_rev, W1_rev, b1_rev, W2_rev, b2_rev, eps_rev)
    return (out_user, out_item)
